# Initial kernel scaffold; baseline (speedup 1.0000x reference)
import jax
import jax.numpy as jnp
from jax.experimental import pallas as pl


def kernel(ZA, RA, qA, muA, quadA, Ka, ZB, RB, qB, muB, quadB, Kb, e_AB_source, e_AB_target):
    raise NotImplementedError("write your pallas kernel here")


# speedup vs baseline: 147.8067x; 147.8067x over previous
import functools

import jax
import jax.numpy as jnp
from jax import lax
from jax.experimental import pallas as pl
from jax.experimental.pallas import tpu as pltpu
from jax.experimental.pallas import tpu_sc as plsc

AU2ANG = 0.529177210903
OO_AU = 1.0 / AU2ANG
PREF = 627.509

_NC, _NS, _L = 2, 16, 16
_NW = _NC * _NS


def _rsqrt(x):
    yi = 0x5F3759DF - lax.shift_right_logical(plsc.bitcast(x, jnp.int32), 1)
    y = plsc.bitcast(yi, jnp.float32)
    xh = 0.5 * x
    y = y * (1.5 - xh * y * y)
    y = y * (1.5 - xh * y * y)
    y = y * (1.5 - xh * y * y)
    return y


def _edge_energy(ga, gb, ux_a, uy_a, uz_a):
    d2a = ux_a * ux_a + uy_a * uy_a + uz_a * uz_a
    d2a = jnp.maximum(d2a, 1e-10)
    ya = _rsqrt(d2a)
    r = d2a * ya * OO_AU
    oodR = AU2ANG * ya
    ux = ux_a * OO_AU
    uy = uy_a * OO_AU
    uz = uz_a * OO_AU
    r2 = r * r
    oodR2 = oodR * oodR
    oodR3 = oodR2 * oodR
    oodR5 = oodR3 * oodR2

    aZ = ga(0)
    aq = ga(4)
    amx, amy, amz = ga(5), ga(6), ga(7)
    ai = ga(14)
    bZ = gb(0)
    bq = gb(4)
    bmx, bmy, bmz = gb(5), gb(6), gb(7)
    aj = gb(14)

    air = ai * r
    ajr = aj * r
    e1r = jnp.exp(-air)
    e2r = jnp.exp(-ajr)
    a1_2 = ai * ai
    a2_2 = aj * aj
    diff = jnp.abs(ai - aj) > 1e-06
    denom = jnp.where(diff, a2_2 - a1_2, 1.0)
    ood = 1.0 / denom
    Ae1 = jnp.where(diff, a2_2 * ood, 0.0) * e1r
    Be2 = jnp.where(diff, -a1_2 * ood, 0.0) * e2r
    lam1 = jnp.where(diff, 1.0 - Ae1 - Be2,
                     1.0 - (1.0 + 0.5 * air) * e1r)
    lam3 = jnp.where(diff, 1.0 - (1.0 + air) * Ae1 - (1.0 + ajr) * Be2,
                     1.0 - (1.0 + air + 0.5 * a1_2 * r2) * e1r)
    lam5 = jnp.where(
        diff,
        1.0 - (1.0 + air + a1_2 * r2 / 3.0) * Ae1
            - (1.0 + ajr + a2_2 * r2 / 3.0) * Be2,
        1.0 - (1.0 + air + 0.5 * a1_2 * r2 + a1_2 * ai * r2 * r / 6.0) * e1r)
    l1j = 1.0 - e2r
    l3j = 1.0 - (1.0 + ajr) * e2r
    l5j = 1.0 - (1.0 + ajr + a2_2 * r2 / 3.0) * e2r
    l1i = 1.0 - e1r
    l3i = 1.0 - (1.0 + air) * e1r
    l5i = 1.0 - (1.0 + air + a1_2 * r2 / 3.0) * e1r

    mAu = amx * ux + amy * uy + amz * uz
    mBu = bmx * ux + bmy * uy + bmz * uz
    mAmB = amx * bmx + amy * bmy + amz * bmz
    ux2, uy2, uz2 = ux * ux, uy * uy, uz * uz
    uxy, uxz, uyz = ux * uy, ux * uz, uy * uz
    uQAu = (ga(8) * ux2 + ga(9) * uy2 + ga(10) * uz2
            + ga(11) * uxy + ga(12) * uxz + ga(13) * uyz)
    trQA = ga(8) + ga(9) + ga(10)
    uQBu = (gb(8) * ux2 + gb(9) * uy2 + gb(10) * uz2
            + gb(11) * uxy + gb(12) * uxz + gb(13) * uyz)
    trQB = gb(8) + gb(9) + gb(10)

    E_qq = aq * bq * oodR * lam1
    E_qu = -oodR3 * lam3 * (aq * mBu - bq * mAu)
    E_uu = -oodR5 * (3.0 * lam5 * mAu * mBu - r2 * lam3 * mAmB)
    E_qQ = oodR5 * (3.0 * lam5 * (aq * uQBu + bq * uQAu)
                    - r2 * lam3 * (aq * trQB + bq * trQA)) / 3.0
    E_ZZ = aZ * bZ * oodR
    E_ZA_MB = aZ * (bq * oodR * l1j - oodR3 * mBu * l3j
                    + oodR5 * (3.0 * l5j * uQBu - r2 * l3j * trQB) / 3.0)
    E_ZB_MA = bZ * (aq * oodR * l1i + oodR3 * mAu * l3i
                    + oodR5 * (3.0 * l5i * uQAu - r2 * l3i * trQA) / 3.0)
    return PREF * (E_qq + E_qu + E_qQ + E_uu + E_ZZ + E_ZA_MB + E_ZB_MA)


@functools.cache
def _sc_forward(n_nodes, n_edges):
    per_w, rem = divmod(n_edges, _NW)
    assert rem == 0, n_edges
    chunk = 0
    for c in range(16, 2049, 16):
        if per_w % c == 0:
            chunk = c
    assert chunk, per_w
    n_chunks = per_w // chunk
    groups = chunk // _L
    mesh = plsc.VectorSubcoreMesh(core_axis_name="c", subcore_axis_name="s")

    @functools.partial(
        pl.kernel,
        out_type=jax.ShapeDtypeStruct((n_edges,), jnp.float32),
        mesh=mesh,
        scratch_types=[
            pltpu.VMEM((chunk,), jnp.int32),
            pltpu.VMEM((chunk,), jnp.int32),
            pltpu.VMEM((chunk, 16), jnp.float32),
            pltpu.VMEM((chunk, 16), jnp.float32),
            pltpu.VMEM((chunk,), jnp.float32),
            pltpu.SemaphoreType.DMA,
            pltpu.SemaphoreType.DMA,
        ],
        compiler_params=pltpu.CompilerParams(
            needs_layout_passes=False, use_tc_tiling_on_sc=False),
    )
    def body(tabA, tabB, src, tgt, out, srcv, tgtv, rA, rB, outv, semA, semB):
        wid = lax.axis_index("s") * _NC + lax.axis_index("c")
        base = wid * per_w
        ibase = lax.iota(jnp.int32, _L)

        def do_chunk(k, carry):
            off = base + k * chunk
            pltpu.sync_copy(src.at[pl.ds(off, chunk)], srcv)
            pltpu.sync_copy(tgt.at[pl.ds(off, chunk)], tgtv)
            cpA = pltpu.async_copy(tabA.at[srcv], rA, semA)
            cpB = pltpu.async_copy(tabB.at[tgtv], rB, semB)
            cpA.wait()
            cpB.wait()

            def do_group(g, carry2):
                row = g * _L + ibase

                def ga(j):
                    return plsc.load_gather(
                        rA, [row, jnp.full((_L,), j, jnp.int32)])

                def gb(j):
                    return plsc.load_gather(
                        rB, [row, jnp.full((_L,), j, jnp.int32)])

                ux_a = gb(1) - ga(1)
                uy_a = gb(2) - ga(2)
                uz_a = gb(3) - ga(3)
                outv[pl.ds(g * _L, _L)] = _edge_energy(ga, gb, ux_a, uy_a, uz_a)
                return carry2

            lax.fori_loop(0, groups, do_group, 0)
            pltpu.sync_copy(outv, out.at[pl.ds(off, chunk)])
            return carry

        lax.fori_loop(0, n_chunks, do_chunk, 0)

    return body


def _pack_table(Z, R, q, mu, quad, K):
    cols = [
        Z,
        R[:, 0], R[:, 1], R[:, 2],
        q - Z,
        mu[:, 0], mu[:, 1], mu[:, 2],
        quad[:, 0, 0], quad[:, 1, 1], quad[:, 2, 2],
        quad[:, 0, 1] + quad[:, 1, 0],
        quad[:, 0, 2] + quad[:, 2, 0],
        quad[:, 1, 2] + quad[:, 2, 1],
        K,
        jnp.zeros_like(Z),
    ]
    return jnp.stack(cols, axis=1)


def kernel(ZA, RA, qA, muA, quadA, Ka, ZB, RB, qB, muB, quadB, Kb,
           e_AB_source, e_AB_target):
    tabA = _pack_table(ZA, RA, qA, muA, quadA, Ka)
    tabB = _pack_table(ZB, RB, qB, muB, quadB, Kb)
    fwd = _sc_forward(ZA.shape[0], e_AB_source.shape[0])
    return fwd(tabA, tabB, e_AB_source, e_AB_target)
